# Initial kernel scaffold; baseline (speedup 1.0000x reference)
import jax
import jax.numpy as jnp
from jax.experimental import pallas as pl


def kernel(x, W_enc, b_enc, W_dec, b_dec):
    raise NotImplementedError("write your pallas kernel here")


# speedup vs baseline: 18.2703x; 18.2703x over previous
import functools

import jax
import jax.numpy as jnp
from jax.experimental import pallas as pl

N_TOK = 8192
ACT_DIM = 1024
DICT_SIZE = 4096
K = 128
BLK = 256


def _body(x_ref, we_ref, be_ref, wd_ref, bd_ref, rec_ref, enc_ref):
    xc = x_ref[...] - bd_ref[...]
    s = jax.lax.dot_general(
        xc, we_ref[...], (((1,), (1,)), ((), ())),
        preferred_element_type=jnp.float32)
    p = jnp.maximum(s + be_ref[...], 0.0)
    pb = jax.lax.bitcast_convert_type(p, jnp.int32)

    def step(i, lo):
        cand = lo | (jnp.int32(1) << (30 - i))
        cnt = jnp.sum((pb >= cand).astype(jnp.float32), axis=1, keepdims=True)
        return jnp.where(cnt >= K, cand, lo)

    lo = jax.lax.fori_loop(0, 31, step, jnp.zeros((BLK, 1), jnp.int32))
    enc = jnp.where(pb >= lo, p, 0.0)
    enc_ref[...] = enc
    rec = jax.lax.dot_general(
        enc, wd_ref[...], (((1,), (1,)), ((), ())),
        preferred_element_type=jnp.float32)
    rec_ref[...] = rec + bd_ref[...]


@functools.partial(jax.jit, static_argnames=("interpret",))
def kernel(x, W_enc, b_enc, W_dec, b_dec, interpret=False):
    n, d = x.shape
    f = W_enc.shape[0]
    grid = (n // BLK,)
    rec, enc = pl.pallas_call(
        _body,
        grid=grid,
        in_specs=[
            pl.BlockSpec((BLK, d), lambda i: (i, 0)),
            pl.BlockSpec((f, d), lambda i: (0, 0)),
            pl.BlockSpec((1, f), lambda i: (0, 0)),
            pl.BlockSpec((d, f), lambda i: (0, 0)),
            pl.BlockSpec((1, d), lambda i: (0, 0)),
        ],
        out_specs=[
            pl.BlockSpec((BLK, d), lambda i: (i, 0)),
            pl.BlockSpec((BLK, f), lambda i: (i, 0)),
        ],
        out_shape=[
            jax.ShapeDtypeStruct((n, d), jnp.float32),
            jax.ShapeDtypeStruct((n, f), jnp.float32),
        ],
        interpret=interpret,
    )(x, W_enc, b_enc.reshape(1, f), W_dec, b_dec.reshape(1, d))
    return (rec, enc)
